# baseline (speedup 1.0000x reference)
import jax
import jax.numpy as jnp
from jax.experimental import pallas as pl


def _body(guT_ref, giT_ref, xui_ref, uT_ref, iT_ref):
    u = guT_ref[...]
    v = giT_ref[...]
    uT_ref[...] = u
    iT_ref[...] = v
    xui_ref[...] = jnp.sum(u * v, axis=0)


def kernel(gu, gi):
    B, D = gu.shape
    BS = 16384
    guT = gu.T
    giT = gi.T
    xui, gamma_uT, gamma_iT = pl.pallas_call(
        _body,
        grid=(B // BS,),
        in_specs=[
            pl.BlockSpec((D, BS), lambda b: (0, b)),
            pl.BlockSpec((D, BS), lambda b: (0, b)),
        ],
        out_specs=[
            pl.BlockSpec((BS,), lambda b: (b,)),
            pl.BlockSpec((D, BS), lambda b: (0, b)),
            pl.BlockSpec((D, BS), lambda b: (0, b)),
        ],
        out_shape=[
            jax.ShapeDtypeStruct((B,), gu.dtype),
            jax.ShapeDtypeStruct((D, B), gu.dtype),
            jax.ShapeDtypeStruct((D, B), gi.dtype),
        ],
    )(guT, giT)
    return (xui, gamma_uT.T, gamma_iT.T)


# speedup vs baseline: 1.1366x; 1.1366x over previous
import jax
import jax.numpy as jnp
from jax.experimental import pallas as pl


def _body(guT_ref, giT_ref, xui_ref, uT_ref, iT_ref):
    u = guT_ref[...]
    v = giT_ref[...]
    uT_ref[...] = u
    iT_ref[...] = v
    xui_ref[...] = jnp.sum(u * v, axis=0)


def kernel(gu, gi):
    B, D = gu.shape
    BS = 8192
    guT = gu.T
    giT = gi.T
    xui, gamma_uT, gamma_iT = pl.pallas_call(
        _body,
        grid=(B // BS,),
        in_specs=[
            pl.BlockSpec((D, BS), lambda b: (0, b)),
            pl.BlockSpec((D, BS), lambda b: (0, b)),
        ],
        out_specs=[
            pl.BlockSpec((BS,), lambda b: (b,)),
            pl.BlockSpec((D, BS), lambda b: (0, b)),
            pl.BlockSpec((D, BS), lambda b: (0, b)),
        ],
        out_shape=[
            jax.ShapeDtypeStruct((B,), gu.dtype),
            jax.ShapeDtypeStruct((D, B), gu.dtype),
            jax.ShapeDtypeStruct((D, B), gi.dtype),
        ],
    )(guT, giT)
    return (xui, gamma_uT.T, gamma_iT.T)
